# baseline (speedup 1.0000x reference)
import functools
import jax
import jax.numpy as jnp
from jax.experimental import pallas as pl

N_NODE = 10000
D = 128
W_SELF = 0.8
W_SEEK = 0.2
N_LAYERS = 3

_BLK = 1000


def _axpy_body(a_ref, b_ref, o_ref):
    o_ref[...] = a_ref[...] + b_ref[...]


def _axpy(a, b):
    grid = (N_NODE // _BLK,)
    return pl.pallas_call(
        _axpy_body,
        grid=grid,
        in_specs=[
            pl.BlockSpec((_BLK, D), lambda i: (i, 0)),
            pl.BlockSpec((_BLK, D), lambda i: (i, 0)),
        ],
        out_specs=pl.BlockSpec((_BLK, D), lambda i: (i, 0)),
        out_shape=jax.ShapeDtypeStruct((N_NODE, D), jnp.float32),
    )(a, b)


def _norm(deg):
    deg = deg.astype(jnp.float32)
    return jnp.where(deg > 0, jax.lax.rsqrt(jnp.maximum(deg, 1.0)), 0.0)


def _segsum(msgs, dst):
    return jax.ops.segment_sum(msgs, dst, num_segments=N_NODE)


def kernel(user_embedding, item_embedding, edge_user_main, edge_game_main,
           edge_user_seek, edge_game_seek, edge_user_dn, edge_game_dn,
           weight_edge):
    n_mu = _norm(jnp.bincount(edge_user_main, length=N_NODE))
    n_mg = _norm(jnp.bincount(edge_game_main, length=N_NODE))
    n_su = _norm(jnp.bincount(edge_user_seek, length=N_NODE))
    n_sg = _norm(jnp.bincount(edge_game_seek, length=N_NODE))
    n_du = _norm(jnp.bincount(edge_user_dn, length=N_NODE))
    n_dg = _norm(jnp.bincount(edge_game_dn, length=N_NODE))

    c_m = n_mu[edge_user_main] * n_mg[edge_game_main]
    c_s = n_su[edge_user_seek] * n_sg[edge_game_seek]
    c_s_w = c_s * weight_edge
    c_d = n_du[edge_user_dn] * n_dg[edge_game_dn]

    dst_u = jnp.concatenate([edge_user_main, edge_user_seek])
    src_g = jnp.concatenate([edge_game_main, edge_game_seek])
    coef_u = jnp.concatenate([W_SELF * c_m, W_SEEK * c_s_w])
    dst_g = jnp.concatenate([edge_game_main, edge_game_seek])
    src_u = jnp.concatenate([edge_user_main, edge_user_seek])
    coef_g = jnp.concatenate([W_SELF * c_m, W_SEEK * c_s])

    h_user = user_embedding
    h_game = item_embedding
    for _ in range(N_LAYERS):
        new_u = _segsum(coef_u[:, None] * h_game[src_g], dst_u)
        new_g = _segsum(coef_g[:, None] * h_user[src_u], dst_g)
        h_user, h_game = new_u, new_g

    hu_s_c = _segsum((W_SEEK * c_s_w)[:, None] * h_game[edge_game_seek],
                     edge_user_seek)
    hi_s_c = _segsum((W_SEEK * c_s)[:, None] * h_user[edge_user_seek],
                     edge_game_seek)

    h1_user = user_embedding
    h1_game = item_embedding
    cd_w = W_SELF * c_d
    for _ in range(N_LAYERS):
        new_u = _axpy(_segsum(cd_w[:, None] * h1_game[edge_game_dn],
                              edge_user_dn), hu_s_c)
        new_g = _axpy(_segsum(cd_w[:, None] * h1_user[edge_user_dn],
                              edge_game_dn), hi_s_c)
        h1_user, h1_game = new_u, new_g

    return (h_user, h_game, h_user, h_game, h1_user, h1_game)


# speedup vs baseline: 8.1471x; 8.1471x over previous
import functools
import jax
import jax.numpy as jnp
from jax import lax
from jax.experimental import pallas as pl
from jax.experimental.pallas import tpu as pltpu
from jax.experimental.pallas import tpu_sc as plsc

N_NODE = 10000
D = 128
W_SELF = 0.8
W_SEEK = 0.2
N_LAYERS = 3
E = 320000

NC = 2
NS = 16
B = 80
NB = E // (NS * B)
G = 25
NCH = NB // G
N_PAD = 10240
RPT = N_PAD // NS
RCH = 8
DCH = 80


def _pass_body(x_hbm, src_hbm, dst_hbm, ew_hbm, out_hbm,
               src_v, dst_v, ew_v, rows_v, acc, sem,
               *, scaled):
    c = lax.axis_index("c")
    s = lax.axis_index("s")
    zeros16 = jnp.zeros((16,), jnp.float32)

    def zero_row(i, cr):
        for j in range(8):
            rows_v[i, pl.ds(16 * j, 16)] = zeros16
        return cr
    lax.fori_loop(0, B, zero_row, 0)
    for k in range(RCH):
        pltpu.sync_copy(rows_v, acc.at[pl.ds(s * RPT + k * DCH, DCH), :])

    plsc.subcore_barrier()

    def chunk_body(ch, carry):
        pltpu.sync_copy(src_hbm.at[c, s, ch], src_v)
        pltpu.sync_copy(dst_hbm.at[c, s, ch], dst_v)
        if scaled:
            pltpu.sync_copy(ew_hbm.at[c, s, ch], ew_v)

        def batch(b2, carry2):
            pltpu.async_copy(x_hbm.at[src_v.at[b2]], rows_v, sem).wait()

            if scaled:
                def scale(e, carry3):
                    ewc = ew_v[b2 * (B // 16) + (e // 16), :]
                    w = lax.gather(
                        ewc, jnp.full((16, 1), e % 16, jnp.int32),
                        dimension_numbers=lax.GatherDimensionNumbers(
                            offset_dims=(), collapsed_slice_dims=(0,),
                            start_index_map=(0,)),
                        slice_sizes=(1,),
                        mode=lax.GatherScatterMode.PROMISE_IN_BOUNDS)
                    for j in range(8):
                        rows_v[e, pl.ds(16 * j, 16)] = (
                            rows_v[e, pl.ds(16 * j, 16)] * w)
                    return carry3
                lax.fori_loop(0, B, scale, 0)

            pltpu.sync_copy(rows_v, acc.at[dst_v.at[b2]], add=True)
            return carry2
        lax.fori_loop(0, G, batch, 0)
        return carry
    lax.fori_loop(0, NCH, chunk_body, 0)

    plsc.subcore_barrier()

    for k in range(RCH):
        r0 = s * RPT + k * DCH
        pltpu.sync_copy(acc.at[pl.ds(r0, DCH), :], rows_v)
        pltpu.sync_copy(rows_v, out_hbm.at[c, pl.ds(r0, DCH), :])


def _make_pass(scaled):
    return pl.kernel(
        functools.partial(_pass_body, scaled=scaled),
        out_type=jax.ShapeDtypeStruct((NC, N_PAD, D), jnp.float32),
        mesh=plsc.VectorSubcoreMesh(core_axis_name="c", subcore_axis_name="s",
                                    num_cores=NC, num_subcores=NS),
        scratch_types=[
            pltpu.VMEM((G, B), jnp.int32),
            pltpu.VMEM((G, B), jnp.int32),
            pltpu.VMEM((G * (B // 16), 16), jnp.float32),
            pltpu.VMEM((B, D), jnp.float32),
            pltpu.VMEM_SHARED((N_PAD, D), jnp.float32),
            pltpu.SemaphoreType.DMA,
        ],
    )


_sc_pass_plain = _make_pass(False)
_sc_pass_w = _make_pass(True)


def _r(a):
    return a.reshape(NS, NCH, G, B)


def _norm(deg):
    deg = deg.astype(jnp.float32)
    return jnp.where(deg > 0, lax.rsqrt(jnp.maximum(deg, 1.0)), 0.0)[:, None]


def kernel(user_embedding, item_embedding, edge_user_main, edge_game_main,
           edge_user_seek, edge_game_seek, edge_user_dn, edge_game_dn,
           weight_edge):
    n_mu = _norm(jnp.bincount(edge_user_main, length=N_NODE))
    n_mg = _norm(jnp.bincount(edge_game_main, length=N_NODE))
    n_su = _norm(jnp.bincount(edge_user_seek, length=N_NODE))
    n_sg = _norm(jnp.bincount(edge_game_seek, length=N_NODE))
    n_du = _norm(jnp.bincount(edge_user_dn, length=N_NODE))
    n_dg = _norm(jnp.bincount(edge_game_dn, length=N_NODE))

    we_w = weight_edge.reshape(NS, NCH, G * (B // 16), 16)
    ones_w = jnp.ones_like(we_w)
    zeros_w = jnp.zeros_like(we_w)

    eum, egm = _r(edge_user_main), _r(edge_game_main)
    eus, egs = _r(edge_user_seek), _r(edge_game_seek)
    eud, egd = _r(edge_user_dn), _r(edge_game_dn)

    def st(a, b):
        return jnp.stack([a, b])

    def st_src(a, b):
        return jnp.stack([a, b + N_NODE])

    def xcat(xa, xb):
        return jnp.concatenate([xa, xb])

    h_user = user_embedding
    h_game = item_embedding
    src_u = st_src(egm, egs)
    dst_u = st(eum, eus)
    ew_u = st(ones_w, we_w)
    src_g = st_src(eum, eus)
    dst_g = st(egm, egs)
    ew_z = st(zeros_w, zeros_w)
    for _ in range(N_LAYERS):
        p_u = _sc_pass_w(xcat(n_mg * h_game, n_sg * h_game), src_u, dst_u,
                         ew_u)
        p_g = _sc_pass_plain(xcat(n_mu * h_user, n_su * h_user), src_g,
                             dst_g, ew_z)
        h_user = W_SELF * n_mu * p_u[0, :N_NODE] + W_SEEK * n_su * p_u[1, :N_NODE]
        h_game = W_SELF * n_mg * p_g[0, :N_NODE] + W_SEEK * n_sg * p_g[1, :N_NODE]

    p_c = _sc_pass_w(xcat(n_sg * h_game, n_su * h_user), st_src(egs, eus),
                     st(eus, egs), st(we_w, ones_w))
    hu_s_c = W_SEEK * n_su * p_c[0, :N_NODE]
    hi_s_c = W_SEEK * n_sg * p_c[1, :N_NODE]

    h1_user = user_embedding
    h1_game = item_embedding
    src_d = st_src(egd, eud)
    dst_d = st(eud, egd)
    for _ in range(N_LAYERS):
        p = _sc_pass_plain(xcat(n_dg * h1_game, n_du * h1_user), src_d,
                           dst_d, ew_z)
        h1_user = W_SELF * n_du * p[0, :N_NODE] + hu_s_c
        h1_game = W_SELF * n_dg * p[1, :N_NODE] + hi_s_c

    return (h_user, h_game, h_user, h_game, h1_user, h1_game)


# speedup vs baseline: 12.1294x; 1.4888x over previous
import functools
import jax
import jax.numpy as jnp
from jax import lax
from jax.experimental import pallas as pl
from jax.experimental.pallas import tpu as pltpu
from jax.experimental.pallas import tpu_sc as plsc

N_NODE = 10000
D = 128
W_SELF = 0.8
W_SEEK = 0.2
N_LAYERS = 3
E = 320000

NC = 2
NS = 16
B = 80
NB = E // (NS * B)
G = 50
NCH = NB // G
N_PAD = 10240
RPT = N_PAD // NS
RCH = 8
DCH = 80


def _pass_body(x_hbm, src_hbm, dst_hbm, ew_hbm, out_hbm,
               src_v, dst_v, ew_v, rows_v, rows_w, acc, sem,
               *, scaled):
    c = lax.axis_index("c")
    s = lax.axis_index("s")
    zeros16 = jnp.zeros((16,), jnp.float32)

    def zero_row(i, cr):
        for j in range(8):
            rows_v[i, pl.ds(16 * j, 16)] = zeros16
        return cr
    lax.fori_loop(0, B, zero_row, 0)
    for k in range(RCH):
        pltpu.sync_copy(rows_v, acc.at[pl.ds(s * RPT + k * DCH, DCH), :])

    plsc.subcore_barrier()

    def _wait_rows(buf):
        pltpu.make_async_copy(x_hbm.at[pl.ds(0, B), :], buf, sem).wait()

    def _process(buf, b2):
        if scaled:
            def scale(e, carry3):
                ewc = ew_v[pl.ds(16 * (b2 * (B // 16) + e // 16), 16)]
                w = lax.gather(
                    ewc, jnp.full((16, 1), e % 16, jnp.int32),
                    dimension_numbers=lax.GatherDimensionNumbers(
                        offset_dims=(), collapsed_slice_dims=(0,),
                        start_index_map=(0,)),
                    slice_sizes=(1,),
                    mode=lax.GatherScatterMode.PROMISE_IN_BOUNDS)
                for j in range(8):
                    buf[e, pl.ds(16 * j, 16)] = buf[e, pl.ds(16 * j, 16)] * w
                return carry3
            lax.fori_loop(0, B, scale, 0)
        pltpu.sync_copy(buf, acc.at[dst_v.at[b2]], add=True)

    def chunk_body(ch, carry):
        pltpu.sync_copy(src_hbm.at[c, s, ch], src_v)
        pltpu.sync_copy(dst_hbm.at[c, s, ch], dst_v)
        if scaled:
            pltpu.sync_copy(ew_hbm.at[c, s, ch], ew_v)

        pltpu.async_copy(x_hbm.at[src_v.at[0]], rows_v, sem)

        def pair(p, carry2):
            pltpu.make_async_copy(x_hbm.at[pl.ds(0, B), :], rows_v, sem).wait()
            pltpu.async_copy(x_hbm.at[src_v.at[2 * p + 1]], rows_w, sem)
            _process(rows_v, 2 * p)
            pltpu.make_async_copy(x_hbm.at[pl.ds(0, B), :], rows_w, sem).wait()
            pltpu.async_copy(x_hbm.at[src_v.at[2 * p + 2]], rows_v, sem)
            _process(rows_w, 2 * p + 1)
            return carry2
        lax.fori_loop(0, G // 2 - 1, pair, 0)

        _wait_rows(rows_v)
        pltpu.async_copy(x_hbm.at[src_v.at[G - 1]], rows_w, sem)
        _process(rows_v, G - 2)
        _wait_rows(rows_w)
        _process(rows_w, G - 1)
        return carry
    lax.fori_loop(0, NCH, chunk_body, 0)

    plsc.subcore_barrier()

    for k in range(RCH):
        r0 = s * RPT + k * DCH
        pltpu.sync_copy(acc.at[pl.ds(r0, DCH), :], rows_v)
        pltpu.sync_copy(rows_v, out_hbm.at[c, pl.ds(r0, DCH), :])


def _make_pass(scaled):
    return pl.kernel(
        functools.partial(_pass_body, scaled=scaled),
        out_type=jax.ShapeDtypeStruct((NC, N_PAD, D), jnp.float32),
        mesh=plsc.VectorSubcoreMesh(core_axis_name="c", subcore_axis_name="s",
                                    num_cores=NC, num_subcores=NS),
        scratch_types=[
            pltpu.VMEM((G, B), jnp.int32),
            pltpu.VMEM((G, B), jnp.int32),
            pltpu.VMEM((G * B,), jnp.float32),
            pltpu.VMEM((B, D), jnp.float32),
            pltpu.VMEM((B, D), jnp.float32),
            pltpu.VMEM_SHARED((N_PAD, D), jnp.float32),
            pltpu.SemaphoreType.DMA,
        ],
    )


_sc_pass_plain = _make_pass(False)
_sc_pass_w = _make_pass(True)


def _r(a):
    return a.reshape(NS, NCH, G, B)


def _norm(deg):
    deg = deg.astype(jnp.float32)
    return jnp.where(deg > 0, lax.rsqrt(jnp.maximum(deg, 1.0)), 0.0)[:, None]


def kernel(user_embedding, item_embedding, edge_user_main, edge_game_main,
           edge_user_seek, edge_game_seek, edge_user_dn, edge_game_dn,
           weight_edge):
    n_mu = _norm(jnp.bincount(edge_user_main, length=N_NODE))
    n_mg = _norm(jnp.bincount(edge_game_main, length=N_NODE))
    n_su = _norm(jnp.bincount(edge_user_seek, length=N_NODE))
    n_sg = _norm(jnp.bincount(edge_game_seek, length=N_NODE))
    n_du = _norm(jnp.bincount(edge_user_dn, length=N_NODE))
    n_dg = _norm(jnp.bincount(edge_game_dn, length=N_NODE))

    we_w = weight_edge.reshape(NS, NCH, G * B)
    ones_w = jnp.ones_like(we_w)
    zeros_w = jnp.zeros_like(we_w)

    eum, egm = _r(edge_user_main), _r(edge_game_main)
    eus, egs = _r(edge_user_seek), _r(edge_game_seek)
    eud, egd = _r(edge_user_dn), _r(edge_game_dn)

    def st(a, b):
        return jnp.stack([a, b])

    def st_src(a, b):
        return jnp.stack([a, b + N_NODE])

    def xcat(xa, xb):
        return jnp.concatenate([xa, xb])

    h_user = user_embedding
    h_game = item_embedding
    src_u = st_src(egm, egs)
    dst_u = st(eum, eus)
    ew_u = st(ones_w, we_w)
    src_g = st_src(eum, eus)
    dst_g = st(egm, egs)
    ew_z = st(zeros_w, zeros_w)
    for _ in range(N_LAYERS):
        p_u = _sc_pass_w(xcat(n_mg * h_game, n_sg * h_game), src_u, dst_u,
                         ew_u)
        p_g = _sc_pass_plain(xcat(n_mu * h_user, n_su * h_user), src_g,
                             dst_g, ew_z)
        h_user = W_SELF * n_mu * p_u[0, :N_NODE] + W_SEEK * n_su * p_u[1, :N_NODE]
        h_game = W_SELF * n_mg * p_g[0, :N_NODE] + W_SEEK * n_sg * p_g[1, :N_NODE]

    p_c = _sc_pass_w(xcat(n_sg * h_game, n_su * h_user), st_src(egs, eus),
                     st(eus, egs), st(we_w, ones_w))
    hu_s_c = W_SEEK * n_su * p_c[0, :N_NODE]
    hi_s_c = W_SEEK * n_sg * p_c[1, :N_NODE]

    h1_user = user_embedding
    h1_game = item_embedding
    src_d = st_src(egd, eud)
    dst_d = st(eud, egd)
    for _ in range(N_LAYERS):
        p = _sc_pass_plain(xcat(n_dg * h1_game, n_du * h1_user), src_d,
                           dst_d, ew_z)
        h1_user = W_SELF * n_du * p[0, :N_NODE] + hu_s_c
        h1_game = W_SELF * n_dg * p[1, :N_NODE] + hi_s_c

    return (h_user, h_game, h_user, h_game, h1_user, h1_game)
